# baseline (speedup 1.0000x reference)
import jax
import jax.numpy as jnp
from jax import lax
from jax.experimental import pallas as pl
from jax.experimental.pallas import tpu as pltpu
from jax.experimental.pallas import tpu_sc as plsc

NC = 2
NS = 16
L = 16
NW = NC * NS

B = 16384
SEQ = 200
D = 10
CB = B // NW
LB = 8
N_CHUNKS = SEQ // LB
BG = CB // L


def _sc_body(xt_hbm, tab_hbm, ot_hbm, tab_v, xb, ob):
    wid = lax.axis_index("s") * NC + lax.axis_index("c")
    b0 = wid * CB

    pltpu.sync_copy(tab_hbm, tab_v.at[pl.ds(0, 100 * D)])
    tab_v[pl.ds(100 * D, L)] = jnp.zeros((L,), jnp.float32)

    @pl.loop(0, N_CHUNKS)
    def _chunk(s):
        l0 = s * LB
        pltpu.sync_copy(xt_hbm.at[pl.ds(l0, LB), pl.ds(b0, CB)], xb)

        @pl.loop(0, BG)
        def _group(g):
            for ll in range(LB):
                xg = xb[ll, pl.ds(g * L, L)]
                a = jnp.where(xg == 0, jnp.int32(100), xg) * D
                for j in range(D):
                    v = plsc.load_gather(tab_v, [a + j])
                    ob[j, ll, pl.ds(g * L, L)] = v

        pltpu.sync_copy(ob, ot_hbm.at[:, pl.ds(l0, LB), pl.ds(b0, CB)])


@jax.jit
def kernel(x, table):
    xt = x.T
    tf = table.reshape(-1)
    call = pl.kernel(
        _sc_body,
        out_type=jax.ShapeDtypeStruct((D, SEQ, B), jnp.float32),
        mesh=plsc.VectorSubcoreMesh(core_axis_name="c", subcore_axis_name="s",
                                    num_cores=NC, num_subcores=NS),
        compiler_params=pltpu.CompilerParams(needs_layout_passes=False),
        scratch_types=[
            pltpu.VMEM((100 * D + L,), jnp.float32),
            pltpu.VMEM((LB, CB), jnp.int32),
            pltpu.VMEM((D, LB, CB), jnp.float32),
        ],
    )
    ot = call(xt, tf)
    return ot.transpose(2, 1, 0)


# speedup vs baseline: 1.2065x; 1.2065x over previous
import jax
import jax.numpy as jnp
from jax import lax
from jax.experimental import pallas as pl
from jax.experimental.pallas import tpu as pltpu
from jax.experimental.pallas import tpu_sc as plsc

NC = 2
NS = 16
L = 16
NW = NC * NS

B = 16384
SEQ = 200
D = 10
CB = B // NW
LB = 8
N_CHUNKS = SEQ // LB
BG = CB // L


def _sc_body(xt_hbm, tab_hbm, ot_hbm, tab_v, xb, ob, sin, sout):
    wid = lax.axis_index("s") * NC + lax.axis_index("c")
    b0 = wid * CB

    def xsrc(c):
        return xt_hbm.at[pl.ds(c * LB, LB), pl.ds(b0, CB)]

    def odst(c):
        return ot_hbm.at[:, pl.ds(c * LB, LB), pl.ds(b0, CB)]

    pltpu.sync_copy(tab_hbm, tab_v.at[pl.ds(0, 100 * D)])
    tab_v[pl.ds(100 * D, L)] = jnp.zeros((L,), jnp.float32)

    pltpu.async_copy(xsrc(0), xb.at[0], sin.at[0])
    pltpu.async_copy(xsrc(1), xb.at[1], sin.at[1])

    def _compute(buf):
        @pl.loop(0, BG)
        def _group(g):
            for ll in range(LB):
                xg = xb[buf, ll, pl.ds(g * L, L)]
                a = jnp.where(xg == 0, jnp.int32(100), xg) * D
                for j in range(D):
                    v = plsc.load_gather(tab_v, [a + j])
                    ob[buf, j, ll, pl.ds(g * L, L)] = v

    def _do_chunk(c, buf):
        pltpu.make_async_copy(xsrc(c), xb.at[buf], sin.at[buf]).wait()

        @pl.when(c >= 2)
        def _():
            pltpu.make_async_copy(ob.at[buf], odst(c - 2), sout.at[buf]).wait()

        _compute(buf)
        pltpu.async_copy(ob.at[buf], odst(c), sout.at[buf])

        @pl.when(c + 2 < N_CHUNKS)
        def _():
            pltpu.async_copy(xsrc(c + 2), xb.at[buf], sin.at[buf])

    @pl.loop(0, N_CHUNKS - 1, step=2)
    def _pair(s):
        _do_chunk(s, 0)
        _do_chunk(s + 1, 1)

    pltpu.make_async_copy(xsrc(N_CHUNKS - 1), xb.at[0], sin.at[0]).wait()
    pltpu.make_async_copy(ob.at[0], odst(N_CHUNKS - 3), sout.at[0]).wait()
    _compute(0)
    pltpu.async_copy(ob.at[0], odst(N_CHUNKS - 1), sout.at[0])
    pltpu.make_async_copy(ob.at[0], odst(N_CHUNKS - 1), sout.at[0]).wait()
    pltpu.make_async_copy(ob.at[1], odst(N_CHUNKS - 2), sout.at[1]).wait()


@jax.jit
def kernel(x, table):
    xt = x.T
    tf = table.reshape(-1)
    call = pl.kernel(
        _sc_body,
        out_type=jax.ShapeDtypeStruct((D, SEQ, B), jnp.float32),
        mesh=plsc.VectorSubcoreMesh(core_axis_name="c", subcore_axis_name="s",
                                    num_cores=NC, num_subcores=NS),
        compiler_params=pltpu.CompilerParams(needs_layout_passes=False),
        scratch_types=[
            pltpu.VMEM((100 * D + L,), jnp.float32),
            pltpu.VMEM((2, LB, CB), jnp.int32),
            pltpu.VMEM((2, D, LB, CB), jnp.float32),
            pltpu.SemaphoreType.DMA((2,)),
            pltpu.SemaphoreType.DMA((2,)),
        ],
    )
    ot = call(xt, tf)
    return ot.transpose(2, 1, 0)


# speedup vs baseline: 2.2407x; 1.8573x over previous
import jax
import jax.numpy as jnp
from jax import lax
from jax.experimental import pallas as pl
from jax.experimental.pallas import tpu as pltpu
from jax.experimental.pallas import tpu_sc as plsc

NC = 2
NS = 16
L = 16
NW = NC * NS

B = 16384
SEQ = 200
D = 10
CB = B // NW
LB = 8
N_CHUNKS = SEQ // LB
BG = CB // L


def _sc_body(xt_hbm, tab_hbm, ot_hbm, tab_v, xb, ob, sin, sout):
    wid = lax.axis_index("s") * NC + lax.axis_index("c")
    b0 = wid * CB

    def xsrc(c):
        return xt_hbm.at[pl.ds(c * LB, LB), pl.ds(b0, CB)]

    def odst(c):
        return ot_hbm.at[:, pl.ds(c * LB, LB), pl.ds(b0, CB)]

    pltpu.sync_copy(tab_hbm, tab_v.at[pl.ds(0, 100 * D)])
    tab_v[pl.ds(100 * D, L)] = jnp.zeros((L,), jnp.float32)

    pltpu.async_copy(xsrc(0), xb.at[0], sin.at[0])
    pltpu.async_copy(xsrc(1), xb.at[1], sin.at[1])

    def _compute(buf):
        @plsc.parallel_loop(0, BG, unroll=2)
        def _group(g):
            for ll in range(LB):
                xg = xb[buf, ll, pl.ds(g * L, L)]
                a = jnp.where(xg == 0, jnp.int32(100), xg) * D
                vs = [plsc.load_gather(tab_v, [a + j]) for j in range(D)]
                for j in range(D):
                    ob[buf, j, ll, pl.ds(g * L, L)] = vs[j]

    def _do_chunk(c, buf):
        pltpu.make_async_copy(xsrc(c), xb.at[buf], sin.at[buf]).wait()

        @pl.when(c >= 2)
        def _():
            pltpu.make_async_copy(ob.at[buf], odst(c - 2), sout.at[buf]).wait()

        _compute(buf)
        pltpu.async_copy(ob.at[buf], odst(c), sout.at[buf])

        @pl.when(c + 2 < N_CHUNKS)
        def _():
            pltpu.async_copy(xsrc(c + 2), xb.at[buf], sin.at[buf])

    @pl.loop(0, N_CHUNKS - 1, step=2)
    def _pair(s):
        _do_chunk(s, 0)
        _do_chunk(s + 1, 1)

    pltpu.make_async_copy(xsrc(N_CHUNKS - 1), xb.at[0], sin.at[0]).wait()
    pltpu.make_async_copy(ob.at[0], odst(N_CHUNKS - 3), sout.at[0]).wait()
    _compute(0)
    pltpu.async_copy(ob.at[0], odst(N_CHUNKS - 1), sout.at[0])
    pltpu.make_async_copy(ob.at[0], odst(N_CHUNKS - 1), sout.at[0]).wait()
    pltpu.make_async_copy(ob.at[1], odst(N_CHUNKS - 2), sout.at[1]).wait()


@jax.jit
def kernel(x, table):
    xt = x.T
    tf = table.reshape(-1)
    call = pl.kernel(
        _sc_body,
        out_type=jax.ShapeDtypeStruct((D, SEQ, B), jnp.float32),
        mesh=plsc.VectorSubcoreMesh(core_axis_name="c", subcore_axis_name="s",
                                    num_cores=NC, num_subcores=NS),
        compiler_params=pltpu.CompilerParams(needs_layout_passes=False),
        scratch_types=[
            pltpu.VMEM((100 * D + L,), jnp.float32),
            pltpu.VMEM((2, LB, CB), jnp.int32),
            pltpu.VMEM((2, D, LB, CB), jnp.float32),
            pltpu.SemaphoreType.DMA((2,)),
            pltpu.SemaphoreType.DMA((2,)),
        ],
    )
    ot = call(xt, tf)
    return ot.transpose(2, 1, 0)


# speedup vs baseline: 3.6375x; 1.6233x over previous
import jax
import jax.numpy as jnp
from jax import lax
from jax.experimental import pallas as pl
from jax.experimental.pallas import tpu as pltpu
from jax.experimental.pallas import tpu_sc as plsc

NC = 2
NS = 16
L = 16
NW = NC * NS

B = 16384
SEQ = 200
D = 10
CB = B // NW
LB = 8
N_CHUNKS = SEQ // LB
BG = CB // L


def _sc_body(xt_hbm, tab_hbm, ot_hbm, tab_v, xb, ob, sin, sout):
    wid = lax.axis_index("s") * NC + lax.axis_index("c")
    b0 = wid * CB

    def xsrc(c):
        return xt_hbm.at[pl.ds(c * LB, LB), pl.ds(b0, CB)]

    def odst(c):
        return ot_hbm.at[:, pl.ds(c * LB, LB), pl.ds(b0, CB)]

    pltpu.sync_copy(tab_hbm, tab_v.at[pl.ds(0, 100 * D)])
    tab_v[pl.ds(100 * D, L)] = jnp.zeros((L,), jnp.float32)

    pltpu.async_copy(xsrc(0), xb.at[0], sin.at[0])
    pltpu.async_copy(xsrc(1), xb.at[1], sin.at[1])

    def _compute(buf):
        @plsc.parallel_loop(0, BG, unroll=4)
        def _group(g):
            for ll in range(LB):
                xg = xb[buf, ll, pl.ds(g * L, L)]
                a = jnp.where(xg == 0, jnp.int32(100), xg) * D
                vs = [plsc.load_gather(tab_v, [a + j]) for j in range(D)]
                for j in range(D):
                    ob[buf, j, ll, pl.ds(g * L, L)] = vs[j]

    def _do_chunk(c, buf):
        pltpu.make_async_copy(xsrc(c), xb.at[buf], sin.at[buf]).wait()

        @pl.when(c >= 2)
        def _():
            pltpu.make_async_copy(ob.at[buf], odst(c - 2), sout.at[buf]).wait()

        _compute(buf)
        pltpu.async_copy(ob.at[buf], odst(c), sout.at[buf])

        @pl.when(c + 2 < N_CHUNKS)
        def _():
            pltpu.async_copy(xsrc(c + 2), xb.at[buf], sin.at[buf])

    @pl.loop(0, N_CHUNKS - 1, step=2)
    def _pair(s):
        _do_chunk(s, 0)
        _do_chunk(s + 1, 1)

    pltpu.make_async_copy(xsrc(N_CHUNKS - 1), xb.at[0], sin.at[0]).wait()
    pltpu.make_async_copy(ob.at[0], odst(N_CHUNKS - 3), sout.at[0]).wait()
    _compute(0)
    pltpu.async_copy(ob.at[0], odst(N_CHUNKS - 1), sout.at[0])
    pltpu.make_async_copy(ob.at[0], odst(N_CHUNKS - 1), sout.at[0]).wait()
    pltpu.make_async_copy(ob.at[1], odst(N_CHUNKS - 2), sout.at[1]).wait()


@jax.jit
def kernel(x, table):
    xt = x.T
    tf = table.reshape(-1)
    call = pl.kernel(
        _sc_body,
        out_type=jax.ShapeDtypeStruct((D, SEQ, B), jnp.float32),
        mesh=plsc.VectorSubcoreMesh(core_axis_name="c", subcore_axis_name="s",
                                    num_cores=NC, num_subcores=NS),
        compiler_params=pltpu.CompilerParams(needs_layout_passes=False),
        scratch_types=[
            pltpu.VMEM((100 * D + L,), jnp.float32),
            pltpu.VMEM((2, LB, CB), jnp.int32),
            pltpu.VMEM((2, D, LB, CB), jnp.float32),
            pltpu.SemaphoreType.DMA((2,)),
            pltpu.SemaphoreType.DMA((2,)),
        ],
    )
    ot = call(xt, tf)
    return ot.transpose(2, 1, 0)


# speedup vs baseline: 4.2276x; 1.1622x over previous
import jax
import jax.numpy as jnp
from jax import lax
from jax.experimental import pallas as pl
from jax.experimental.pallas import tpu as pltpu
from jax.experimental.pallas import tpu_sc as plsc

NC = 2
NS = 16
L = 16
NW = NC * NS

B = 16384
SEQ = 200
D = 10
CB = B // NW
LB = 8
N_CHUNKS = SEQ // LB
BG = CB // L


def _sc_body(xt_hbm, tab_hbm, ot_hbm, tab_v, xb, ob, sin, sout):
    wid = lax.axis_index("s") * NC + lax.axis_index("c")
    b0 = wid * CB

    def xsrc(c):
        return xt_hbm.at[pl.ds(c * LB, LB), pl.ds(b0, CB)]

    def odst(c):
        return ot_hbm.at[:, pl.ds(c * LB, LB), pl.ds(b0, CB)]

    pltpu.sync_copy(tab_hbm, tab_v.at[pl.ds(0, 100 * D)])
    tab_v[pl.ds(100 * D, L)] = jnp.zeros((L,), jnp.float32)

    pltpu.async_copy(xsrc(0), xb.at[0], sin.at[0])
    pltpu.async_copy(xsrc(1), xb.at[1], sin.at[1])

    def _compute(buf):
        @plsc.parallel_loop(0, BG, unroll=8)
        def _group(g):
            for ll in range(LB):
                xg = xb[buf, ll, pl.ds(g * L, L)]
                a = jnp.where(xg == 0, jnp.int32(100), xg) * D
                vs = [plsc.load_gather(tab_v, [a + j]) for j in range(D)]
                for j in range(D):
                    ob[buf, j, ll, pl.ds(g * L, L)] = vs[j]

    def _do_chunk(c, buf):
        pltpu.make_async_copy(xsrc(c), xb.at[buf], sin.at[buf]).wait()

        @pl.when(c >= 2)
        def _():
            pltpu.make_async_copy(ob.at[buf], odst(c - 2), sout.at[buf]).wait()

        _compute(buf)
        pltpu.async_copy(ob.at[buf], odst(c), sout.at[buf])

        @pl.when(c + 2 < N_CHUNKS)
        def _():
            pltpu.async_copy(xsrc(c + 2), xb.at[buf], sin.at[buf])

    @pl.loop(0, N_CHUNKS - 1, step=2)
    def _pair(s):
        _do_chunk(s, 0)
        _do_chunk(s + 1, 1)

    pltpu.make_async_copy(xsrc(N_CHUNKS - 1), xb.at[0], sin.at[0]).wait()
    pltpu.make_async_copy(ob.at[0], odst(N_CHUNKS - 3), sout.at[0]).wait()
    _compute(0)
    pltpu.async_copy(ob.at[0], odst(N_CHUNKS - 1), sout.at[0])
    pltpu.make_async_copy(ob.at[0], odst(N_CHUNKS - 1), sout.at[0]).wait()
    pltpu.make_async_copy(ob.at[1], odst(N_CHUNKS - 2), sout.at[1]).wait()


@jax.jit
def kernel(x, table):
    xt = x.T
    tf = table.reshape(-1)
    call = pl.kernel(
        _sc_body,
        out_type=jax.ShapeDtypeStruct((D, SEQ, B), jnp.float32),
        mesh=plsc.VectorSubcoreMesh(core_axis_name="c", subcore_axis_name="s",
                                    num_cores=NC, num_subcores=NS),
        compiler_params=pltpu.CompilerParams(needs_layout_passes=False),
        scratch_types=[
            pltpu.VMEM((100 * D + L,), jnp.float32),
            pltpu.VMEM((2, LB, CB), jnp.int32),
            pltpu.VMEM((2, D, LB, CB), jnp.float32),
            pltpu.SemaphoreType.DMA((2,)),
            pltpu.SemaphoreType.DMA((2,)),
        ],
    )
    ot = call(xt, tf)
    return ot.transpose(2, 1, 0)
